# initial kernel scaffold (unmeasured)
import jax
import jax.numpy as jnp
from jax import lax
from jax.experimental import pallas as pl
from jax.experimental.pallas import tpu as pltpu


def kernel(
    x,
):
    def body(*refs):
        pass

    out_shape = jax.ShapeDtypeStruct(..., jnp.float32)
    return pl.pallas_call(body, out_shape=out_shape)(...)


# baseline (device time: 203079 ns/iter reference)
import jax
import jax.numpy as jnp
from jax import lax
from jax.experimental import pallas as pl
from jax.experimental.pallas import tpu as pltpu


def kernel(x):
    m, n = x.shape

    def body(x_ref, out_ref, send_sem, recv_sem):
        my_x = lax.axis_index("x")
        my_y = lax.axis_index("y")
        my_z = lax.axis_index("z")

        rdma = pltpu.make_async_remote_copy(
            src_ref=x_ref,
            dst_ref=out_ref,
            send_sem=send_sem,
            recv_sem=recv_sem,
            device_id=(my_x, 1 - my_y, my_z),
            device_id_type=pl.DeviceIdType.MESH,
        )
        rdma.start()
        rdma.wait()
        out_ref[...] = out_ref[...] + x_ref[...]

    return pl.pallas_call(
        body,
        out_shape=jax.ShapeDtypeStruct((m, n), x.dtype),
        in_specs=[pl.BlockSpec(memory_space=pltpu.VMEM)],
        out_specs=pl.BlockSpec(memory_space=pltpu.VMEM),
        scratch_shapes=[
            pltpu.SemaphoreType.DMA,
            pltpu.SemaphoreType.DMA,
        ],
    )(x)


# device time: 113532 ns/iter; 1.7887x vs baseline; 1.7887x over previous
import jax
import jax.numpy as jnp
from jax import lax
from jax.experimental import pallas as pl
from jax.experimental.pallas import tpu as pltpu

NC = 16


def kernel(x):
    m, n = x.shape
    half = m // 2
    r = half // NC

    def body(x_ref, out_ref, yrecv, ysend_sem, yrecv_sem, xsend_sem, xrecv_sem):
        my_x = lax.axis_index("x")
        my_y = lax.axis_index("y")
        my_z = lax.axis_index("z")
        y_nbr = (my_x, 1 - my_y, my_z)
        x_nbr = (1 - my_x, my_y, my_z)
        base = my_x * half

        barrier_sem = pltpu.get_barrier_semaphore()
        for nbr in (y_nbr, x_nbr):
            pl.semaphore_signal(
                barrier_sem, inc=1,
                device_id=nbr, device_id_type=pl.DeviceIdType.MESH,
            )
        pl.semaphore_wait(barrier_sem, 2)

        y_rdmas = []
        for i in range(NC):
            rdma = pltpu.make_async_remote_copy(
                src_ref=x_ref.at[pl.ds(base + i * r, r), :],
                dst_ref=yrecv.at[i],
                send_sem=ysend_sem.at[i],
                recv_sem=yrecv_sem.at[i],
                device_id=y_nbr,
                device_id_type=pl.DeviceIdType.MESH,
            )
            rdma.start()
            y_rdmas.append(rdma)

        x_rdmas = []
        for i in range(NC):
            y_rdmas[i].wait_recv()
            rows = pl.ds(base + i * r, r)
            out_ref[rows, :] = x_ref[rows, :] + yrecv[i, :, :]
            rdma = pltpu.make_async_remote_copy(
                src_ref=out_ref.at[rows, :],
                dst_ref=out_ref.at[rows, :],
                send_sem=xsend_sem.at[i],
                recv_sem=xrecv_sem.at[i],
                device_id=x_nbr,
                device_id_type=pl.DeviceIdType.MESH,
            )
            rdma.start()
            x_rdmas.append(rdma)

        for i in range(NC):
            x_rdmas[i].wait_recv()
        for i in range(NC):
            y_rdmas[i].wait_send()
            x_rdmas[i].wait_send()

    return pl.pallas_call(
        body,
        out_shape=jax.ShapeDtypeStruct((m, n), x.dtype),
        in_specs=[pl.BlockSpec(memory_space=pltpu.VMEM)],
        out_specs=pl.BlockSpec(memory_space=pltpu.VMEM),
        scratch_shapes=[
            pltpu.VMEM((NC, r, n), x.dtype),
            pltpu.SemaphoreType.DMA((NC,)),
            pltpu.SemaphoreType.DMA((NC,)),
            pltpu.SemaphoreType.DMA((NC,)),
            pltpu.SemaphoreType.DMA((NC,)),
        ],
        compiler_params=pltpu.CompilerParams(collective_id=0),
    )(x)


# device time: 111220 ns/iter; 1.8259x vs baseline; 1.0208x over previous
import jax
import jax.numpy as jnp
from jax import lax
from jax.experimental import pallas as pl
from jax.experimental.pallas import tpu as pltpu

NC = 32


def kernel(x):
    m, n = x.shape
    half = m // 2
    r = half // NC

    def body(x_ref, out_ref, yrecv, ysend_sem, yrecv_sem, xsend_sem, xrecv_sem):
        my_x = lax.axis_index("x")
        my_y = lax.axis_index("y")
        my_z = lax.axis_index("z")
        y_nbr = (my_x, 1 - my_y, my_z)
        x_nbr = (1 - my_x, my_y, my_z)
        base = my_x * half

        barrier_sem = pltpu.get_barrier_semaphore()
        for nbr in (y_nbr, x_nbr):
            pl.semaphore_signal(
                barrier_sem, inc=1,
                device_id=nbr, device_id_type=pl.DeviceIdType.MESH,
            )
        pl.semaphore_wait(barrier_sem, 2)

        y_rdmas = []
        for i in range(NC):
            rdma = pltpu.make_async_remote_copy(
                src_ref=x_ref.at[pl.ds(base + i * r, r), :],
                dst_ref=yrecv.at[i],
                send_sem=ysend_sem.at[i],
                recv_sem=yrecv_sem.at[i],
                device_id=y_nbr,
                device_id_type=pl.DeviceIdType.MESH,
            )
            rdma.start()
            y_rdmas.append(rdma)

        x_rdmas = []
        for i in range(NC):
            y_rdmas[i].wait_recv()
            rows = pl.ds(base + i * r, r)
            rdma = pltpu.make_async_remote_copy(
                src_ref=yrecv.at[i],
                dst_ref=out_ref.at[rows, :],
                send_sem=xsend_sem.at[i],
                recv_sem=xrecv_sem.at[i],
                device_id=x_nbr,
                device_id_type=pl.DeviceIdType.MESH,
            )
            rdma.start()
            out_ref[rows, :] = x_ref[rows, :] + yrecv[i, :, :]
            x_rdmas.append(rdma)

        obase = half - base
        for i in range(NC):
            x_rdmas[i].wait_recv()
            orows = pl.ds(obase + i * r, r)
            out_ref[orows, :] = out_ref[orows, :] + x_ref[orows, :]
        for i in range(NC):
            y_rdmas[i].wait_send()
            x_rdmas[i].wait_send()

    return pl.pallas_call(
        body,
        out_shape=jax.ShapeDtypeStruct((m, n), x.dtype),
        in_specs=[pl.BlockSpec(memory_space=pltpu.VMEM)],
        out_specs=pl.BlockSpec(memory_space=pltpu.VMEM),
        scratch_shapes=[
            pltpu.VMEM((NC, r, n), x.dtype),
            pltpu.SemaphoreType.DMA((NC,)),
            pltpu.SemaphoreType.DMA((NC,)),
            pltpu.SemaphoreType.DMA((NC,)),
            pltpu.SemaphoreType.DMA((NC,)),
        ],
        compiler_params=pltpu.CompilerParams(collective_id=0),
    )(x)
